# baseline (speedup 1.0000x reference)
import functools

import jax
import jax.numpy as jnp
from jax import lax
from jax.experimental import pallas as pl
from jax.experimental.pallas import tpu as pltpu
from jax.experimental.pallas import tpu_sc as plsc

N = 10000
NP = 10240
E = 320000
D = 128
NC = 2
NS = 16
NW = NC * NS
EW = E // NW
RT = NP // NS

KA = 128
EP = NW * 10240
CA = EP // NW // KA
CH = CA // 2

KC = 50
CC = EW // KC

CW = 16

_MESH = plsc.VectorSubcoreMesh(core_axis_name="c", subcore_axis_name="s")


@functools.partial(
    pl.kernel,
    out_type=jax.ShapeDtypeStruct((NC, NP, D), jnp.float32),
    mesh=_MESH,
    scratch_types=[
        pltpu.VMEM((CH, KA), jnp.int32),
        pltpu.VMEM((CH, KA), jnp.int32),
        pltpu.VMEM((KA, D), jnp.float32),
        pltpu.VMEM((KA, D), jnp.float32),
        pltpu.VMEM_SHARED((NP, D), jnp.float32),
        pltpu.SemaphoreType.DMA,
        pltpu.SemaphoreType.DMA,
    ],
)
def _sc_aggregate(h_hbm, src_hbm, dst_hbm, zeros_hbm, out_hbm,
                  idx_s, idx_d, msg0, msg1, acc, sem0, sem1):
    c = lax.axis_index("c")
    s = lax.axis_index("s")
    wid = s * NC + c
    pltpu.sync_copy(zeros_hbm, acc.at[pl.ds(s * RT, RT)])
    plsc.subcore_barrier()

    for half in range(2):
        pltpu.sync_copy(src_hbm.at[wid, pl.ds(half * CH, CH)], idx_s)
        pltpu.sync_copy(dst_hbm.at[wid, pl.ds(half * CH, CH)], idx_d)

        pltpu.async_copy(h_hbm.at[idx_s.at[0]], msg0, sem0)
        pltpu.async_copy(h_hbm.at[idx_s.at[1]], msg1, sem1)

        def body(i, carry):
            j0 = 2 * i
            pltpu.make_async_copy(h_hbm.at[idx_s.at[j0]], msg0, sem0).wait()
            pltpu.sync_copy(msg0, acc.at[idx_d.at[j0]], add=True)
            pltpu.async_copy(h_hbm.at[idx_s.at[j0 + 2]], msg0, sem0)

            pltpu.make_async_copy(h_hbm.at[idx_s.at[j0 + 1]], msg1, sem1).wait()
            pltpu.sync_copy(msg1, acc.at[idx_d.at[j0 + 1]], add=True)
            pltpu.async_copy(h_hbm.at[idx_s.at[j0 + 3]], msg1, sem1)
            return carry

        lax.fori_loop(0, CH // 2 - 1, body, 0)
        j0 = CH - 2
        pltpu.make_async_copy(h_hbm.at[idx_s.at[j0]], msg0, sem0).wait()
        pltpu.sync_copy(msg0, acc.at[idx_d.at[j0]], add=True)
        pltpu.make_async_copy(h_hbm.at[idx_s.at[j0 + 1]], msg1, sem1).wait()
        pltpu.sync_copy(msg1, acc.at[idx_d.at[j0 + 1]], add=True)

    plsc.subcore_barrier()
    pltpu.sync_copy(acc.at[pl.ds(s * RT, RT)], out_hbm.at[c, pl.ds(s * RT, RT)])


@functools.partial(
    pl.kernel,
    out_type=jax.ShapeDtypeStruct((NC, NP, CW), jnp.float32),
    mesh=_MESH,
    scratch_types=[
        pltpu.VMEM((CC, KC), jnp.int32),
        pltpu.VMEM((KC, CW), jnp.float32),
        pltpu.VMEM_SHARED((NP, CW), jnp.float32),
    ],
    compiler_params=pltpu.CompilerParams(use_tc_tiling_on_sc=False),
)
def _sc_count(dst_hbm, zeros_hbm, ones_hbm, out_hbm, idx_d, ones_v, acc):
    c = lax.axis_index("c")
    s = lax.axis_index("s")
    wid = s * NC + c
    pltpu.sync_copy(zeros_hbm, acc.at[pl.ds(s * RT, RT)])
    pltpu.sync_copy(ones_hbm, ones_v)
    pltpu.sync_copy(dst_hbm.at[wid], idx_d)
    plsc.subcore_barrier()

    def body(j, carry):
        pltpu.sync_copy(ones_v, acc.at[idx_d.at[j]], add=True)
        return carry

    lax.fori_loop(0, CC, body, 0)
    plsc.subcore_barrier()
    pltpu.sync_copy(acc.at[pl.ds(s * RT, RT)], out_hbm.at[c, pl.ds(s * RT, RT)])


BR = 1024
GB = NP // BR


def _dense_body(aggp_ref, cntp_ref, h_ref, wl_ref, wr_ref, b_ref, o_ref, *, act):
    cnt = cntp_ref[0, :, 0:1] + cntp_ref[1, :, 0:1]
    inv = 1.0 / jnp.maximum(cnt, 1.0)
    agg = (aggp_ref[0] + aggp_ref[1]) * inv
    out = (jnp.dot(agg, wl_ref[...], preferred_element_type=jnp.float32)
           + jnp.dot(h_ref[...], wr_ref[...], preferred_element_type=jnp.float32)
           + b_ref[...])
    if act == "relu":
        out = jnp.maximum(out, 0.0)
    else:
        z = out - jnp.max(out, axis=-1, keepdims=True)
        out = z - jnp.log(jnp.sum(jnp.exp(z), axis=-1, keepdims=True))
    o_ref[...] = out


def _dense(aggp, cntp, h, Wl, Wr, b2d, act):
    return pl.pallas_call(
        functools.partial(_dense_body, act=act),
        grid=(GB,),
        in_specs=[
            pl.BlockSpec((2, BR, D), lambda i: (0, i, 0)),
            pl.BlockSpec((2, BR, CW), lambda i: (0, i, 0)),
            pl.BlockSpec((BR, D), lambda i: (i, 0)),
            pl.BlockSpec((D, D), lambda i: (0, 0)),
            pl.BlockSpec((D, D), lambda i: (0, 0)),
            pl.BlockSpec((1, D), lambda i: (0, 0)),
        ],
        out_specs=pl.BlockSpec((BR, D), lambda i: (i, 0)),
        out_shape=jax.ShapeDtypeStruct((NP, D), jnp.float32),
    )(aggp, cntp, h, Wl, Wr, b2d)


def kernel(x, edge_index, Wl1, Wr1, b1, Wl2, Wr2, b2, Wl3, Wr3, b3):
    pad = EP - E
    src3 = jnp.concatenate(
        [edge_index[0], jnp.zeros((pad,), jnp.int32)]).reshape(NW, CA, KA)
    dst3 = jnp.concatenate(
        [edge_index[1], jnp.full((pad,), N, jnp.int32)]).reshape(NW, CA, KA)
    dstC = edge_index[1].reshape(NW, CC, KC)
    xp = jnp.zeros((NP, D), jnp.float32).at[:N].set(x)
    zrows = jnp.zeros((RT, D), jnp.float32)
    zc = jnp.zeros((RT, CW), jnp.float32)
    ones = jnp.ones((KC, CW), jnp.float32)

    cntp = _sc_count(dstC, zc, ones)
    h = xp
    for Wl, Wr, b, act in ((Wl1, Wr1, b1, "relu"),
                           (Wl2, Wr2, b2, "relu"),
                           (Wl3, Wr3, b3, "logsoftmax")):
        aggp = _sc_aggregate(h, src3, dst3, zrows)
        h = _dense(aggp, cntp, h, Wl, Wr, b.reshape(1, D), act)
    return h[:N]


# speedup vs baseline: 1.0711x; 1.0711x over previous
import functools

import jax
import jax.numpy as jnp
from jax import lax
from jax.experimental import pallas as pl
from jax.experimental.pallas import tpu as pltpu
from jax.experimental.pallas import tpu_sc as plsc

N = 10000
NP = 10240
E = 320000
D = 128
NC = 2
NS = 16
NW = NC * NS
EW = E // NW
RT = NP // NS

KA = 128
EP = NW * 10240
CA = EP // NW // KA
CH = CA // 2

KC = 50
CC = EW // KC

CW = 16

_MESH = plsc.VectorSubcoreMesh(core_axis_name="c", subcore_axis_name="s")


@functools.partial(
    pl.kernel,
    out_type=jax.ShapeDtypeStruct((NC, NP, D), jnp.float32),
    mesh=_MESH,
    scratch_types=[
        pltpu.VMEM((CH, KA), jnp.int32),
        pltpu.VMEM((CH, KA), jnp.int32),
        pltpu.VMEM((KA, D), jnp.float32),
        pltpu.VMEM((KA, D), jnp.float32),
        pltpu.VMEM_SHARED((NP, D), jnp.float32),
        pltpu.SemaphoreType.DMA,
        pltpu.SemaphoreType.DMA,
    ],
)
def _sc_aggregate(h_hbm, src_hbm, dst_hbm, zeros_hbm, out_hbm,
                  idx_s, idx_d, msg0, msg1, acc, sem0, sem1):
    c = lax.axis_index("c")
    s = lax.axis_index("s")
    wid = s * NC + c
    pltpu.sync_copy(zeros_hbm, acc.at[pl.ds(s * RT, RT)])
    plsc.subcore_barrier()

    for half in range(2):
        pltpu.sync_copy(src_hbm.at[wid, pl.ds(half * CH, CH)], idx_s)
        pltpu.sync_copy(dst_hbm.at[wid, pl.ds(half * CH, CH)], idx_d)

        pltpu.async_copy(h_hbm.at[idx_s.at[0]], msg0, sem0)
        pltpu.async_copy(h_hbm.at[idx_s.at[1]], msg1, sem1)

        def body(i, carry):
            j0 = 2 * i
            pltpu.make_async_copy(h_hbm.at[idx_s.at[j0]], msg0, sem0).wait()
            pltpu.sync_copy(msg0, acc.at[idx_d.at[j0]], add=True)
            pltpu.async_copy(h_hbm.at[idx_s.at[j0 + 2]], msg0, sem0)

            pltpu.make_async_copy(h_hbm.at[idx_s.at[j0 + 1]], msg1, sem1).wait()
            pltpu.sync_copy(msg1, acc.at[idx_d.at[j0 + 1]], add=True)
            pltpu.async_copy(h_hbm.at[idx_s.at[j0 + 3]], msg1, sem1)
            return carry

        lax.fori_loop(0, CH // 2 - 1, body, 0)
        j0 = CH - 2
        pltpu.make_async_copy(h_hbm.at[idx_s.at[j0]], msg0, sem0).wait()
        pltpu.sync_copy(msg0, acc.at[idx_d.at[j0]], add=True)
        pltpu.make_async_copy(h_hbm.at[idx_s.at[j0 + 1]], msg1, sem1).wait()
        pltpu.sync_copy(msg1, acc.at[idx_d.at[j0 + 1]], add=True)

    plsc.subcore_barrier()
    pltpu.sync_copy(acc.at[pl.ds(s * RT, RT)], out_hbm.at[c, pl.ds(s * RT, RT)])


@functools.partial(
    pl.kernel,
    out_type=jax.ShapeDtypeStruct((NC, NP, CW), jnp.float32),
    mesh=_MESH,
    scratch_types=[
        pltpu.VMEM((CC, KC), jnp.int32),
        pltpu.VMEM((KC, CW), jnp.float32),
        pltpu.VMEM_SHARED((NP, CW), jnp.float32),
    ],
    compiler_params=pltpu.CompilerParams(use_tc_tiling_on_sc=False),
)
def _sc_count(dst_hbm, zeros_hbm, ones_hbm, out_hbm, idx_d, ones_v, acc):
    c = lax.axis_index("c")
    s = lax.axis_index("s")
    wid = s * NC + c
    pltpu.sync_copy(zeros_hbm, acc.at[pl.ds(s * RT, RT)])
    pltpu.sync_copy(ones_hbm, ones_v)
    pltpu.sync_copy(dst_hbm.at[wid], idx_d)
    plsc.subcore_barrier()

    def body(j, carry):
        pltpu.sync_copy(ones_v, acc.at[idx_d.at[j]], add=True)
        return carry

    lax.fori_loop(0, CC, body, 0)
    plsc.subcore_barrier()
    pltpu.sync_copy(acc.at[pl.ds(s * RT, RT)], out_hbm.at[c, pl.ds(s * RT, RT)])


BR = 1024
GB = NP // BR


def _dense_body(aggp_ref, cntp_ref, h_ref, wl_ref, wr_ref, b_ref, o_ref, *, act):
    cnt = cntp_ref[0, :, 0:1] + cntp_ref[1, :, 0:1]
    inv = 1.0 / jnp.maximum(cnt, 1.0)
    agg = (aggp_ref[0] + aggp_ref[1]) * inv
    out = (jnp.dot(agg, wl_ref[...], preferred_element_type=jnp.float32)
           + jnp.dot(h_ref[...], wr_ref[...], preferred_element_type=jnp.float32)
           + b_ref[...])
    if act == "relu":
        out = jnp.maximum(out, 0.0)
    else:
        z = out - jnp.max(out, axis=-1, keepdims=True)
        out = z - jnp.log(jnp.sum(jnp.exp(z), axis=-1, keepdims=True))
    o_ref[...] = out


def _dense(aggp, cntp, h, Wl, Wr, b2d, act):
    return pl.pallas_call(
        functools.partial(_dense_body, act=act),
        grid=(GB,),
        in_specs=[
            pl.BlockSpec((2, BR, D), lambda i: (0, i, 0)),
            pl.BlockSpec((2, BR, CW), lambda i: (0, i, 0)),
            pl.BlockSpec((BR, D), lambda i: (i, 0)),
            pl.BlockSpec((D, D), lambda i: (0, 0)),
            pl.BlockSpec((D, D), lambda i: (0, 0)),
            pl.BlockSpec((1, D), lambda i: (0, 0)),
        ],
        out_specs=pl.BlockSpec((BR, D), lambda i: (i, 0)),
        out_shape=jax.ShapeDtypeStruct((NP, D), jnp.float32),
    )(aggp, cntp, h, Wl, Wr, b2d)


def kernel(x, edge_index, Wl1, Wr1, b1, Wl2, Wr2, b2, Wl3, Wr3, b3):
    pad = EP - E
    src3 = jnp.concatenate(
        [edge_index[0], jnp.zeros((pad,), jnp.int32)]).reshape(NW, CA, KA)
    trash = N + jnp.arange(pad, dtype=jnp.int32) % (NP - N)
    dst3 = jnp.concatenate([edge_index[1], trash]).reshape(NW, CA, KA)
    dstC = edge_index[1].reshape(NW, CC, KC)
    xp = jnp.zeros((NP, D), jnp.float32).at[:N].set(x)
    zrows = jnp.zeros((RT, D), jnp.float32)
    zc = jnp.zeros((RT, CW), jnp.float32)
    ones = jnp.ones((KC, CW), jnp.float32)

    cntp = _sc_count(dstC, zc, ones)
    h = xp
    for Wl, Wr, b, act in ((Wl1, Wr1, b1, "relu"),
                           (Wl2, Wr2, b2, "relu"),
                           (Wl3, Wr3, b3, "logsoftmax")):
        aggp = _sc_aggregate(h, src3, dst3, zrows)
        h = _dense(aggp, cntp, h, Wl, Wr, b.reshape(1, D), act)
    return h[:N]


# speedup vs baseline: 2.5002x; 2.3343x over previous
import functools

import jax
import jax.numpy as jnp
from jax import lax
from jax.experimental import pallas as pl
from jax.experimental.pallas import tpu as pltpu
from jax.experimental.pallas import tpu_sc as plsc

N = 10000
NP = 10240
E = 320000
D = 128
DH = D // 2
NC = 2
NS = 16
NW = NC * NS
RT = NP // NS

KA = 125
CA = E // NS // KA

KC = 50
CC = E // NW // KC

_MESH = plsc.VectorSubcoreMesh(core_axis_name="c", subcore_axis_name="s")


CW = 16


def _make_aggregate(with_count):
    out_type = [jax.ShapeDtypeStruct((NC, NP, DH), jnp.float32)]
    scratch = [
        pltpu.VMEM((CA, KA), jnp.int32),
        pltpu.VMEM((CA, KA), jnp.int32),
        pltpu.VMEM((KA, DH), jnp.float32),
        pltpu.VMEM((KA, DH), jnp.float32),
        pltpu.VMEM_SHARED((NP, DH), jnp.float32),
        pltpu.SemaphoreType.DMA,
        pltpu.SemaphoreType.DMA,
    ]
    if with_count:
        out_type.append(jax.ShapeDtypeStruct((NC, NP, CW), jnp.float32))
        scratch += [
            pltpu.VMEM((KA, CW), jnp.float32),
            pltpu.VMEM_SHARED((NP, CW), jnp.float32),
        ]

    def agg(h_hbm, src_hbm, dst_hbm, zeros_hbm, *rest):
        if with_count:
            (zc_hbm, ones_hbm, out_hbm, cnt_hbm,
             idx_s, idx_d, msg0, msg1, acc, sem0, sem1, ones_v, cacc) = rest
        else:
            idx_s, idx_d, msg0, msg1, acc, sem0, sem1 = rest[1:]
            out_hbm = rest[0]
        c = lax.axis_index("c")
        s = lax.axis_index("s")
        pltpu.sync_copy(zeros_hbm, acc.at[pl.ds(s * RT, RT)])
        pltpu.sync_copy(src_hbm.at[c, s], idx_s)
        pltpu.sync_copy(dst_hbm.at[s], idx_d)
        if with_count:
            pltpu.sync_copy(zc_hbm, cacc.at[pl.ds(s * RT, RT)])
            pltpu.sync_copy(ones_hbm, ones_v)
        plsc.subcore_barrier()

        pltpu.async_copy(h_hbm.at[idx_s.at[0]], msg0, sem0)
        pltpu.async_copy(h_hbm.at[idx_s.at[1]], msg1, sem1)

        def body(i, carry):
            j0 = 2 * i
            pltpu.make_async_copy(h_hbm.at[idx_s.at[j0]], msg0, sem0).wait()
            pltpu.sync_copy(msg0, acc.at[idx_d.at[j0]], add=True)
            if with_count:
                @pl.when(c == 0)
                def _():
                    pltpu.sync_copy(ones_v, cacc.at[idx_d.at[j0]], add=True)

            @pl.when(j0 + 2 < CA)
            def _():
                pltpu.async_copy(h_hbm.at[idx_s.at[j0 + 2]], msg0, sem0)

            pltpu.make_async_copy(h_hbm.at[idx_s.at[j0 + 1]], msg1, sem1).wait()
            pltpu.sync_copy(msg1, acc.at[idx_d.at[j0 + 1]], add=True)
            if with_count:
                @pl.when(c == 1)
                def _():
                    pltpu.sync_copy(ones_v, cacc.at[idx_d.at[j0 + 1]], add=True)

            @pl.when(j0 + 3 < CA)
            def _():
                pltpu.async_copy(h_hbm.at[idx_s.at[j0 + 3]], msg1, sem1)

            return carry

        lax.fori_loop(0, CA // 2, body, 0)
        plsc.subcore_barrier()
        pltpu.sync_copy(acc.at[pl.ds(s * RT, RT)], out_hbm.at[c, pl.ds(s * RT, RT)])
        if with_count:
            pltpu.sync_copy(cacc.at[pl.ds(s * RT, RT)],
                            cnt_hbm.at[c, pl.ds(s * RT, RT)])

    return functools.partial(
        pl.kernel,
        out_type=tuple(out_type) if with_count else out_type[0],
        mesh=_MESH,
        scratch_types=scratch,
        compiler_params=pltpu.CompilerParams(use_tc_tiling_on_sc=False),
    )(agg)


_sc_aggregate = _make_aggregate(False)


@functools.partial(
    pl.kernel,
    out_type=jax.ShapeDtypeStruct((NC, NP, CW), jnp.float32),
    mesh=_MESH,
    scratch_types=[
        pltpu.VMEM((CC, KC), jnp.int32),
        pltpu.VMEM((KC, CW), jnp.float32),
        pltpu.VMEM_SHARED((NP, CW), jnp.float32),
    ],
    compiler_params=pltpu.CompilerParams(use_tc_tiling_on_sc=False),
)
def _sc_count(dst_hbm, zeros_hbm, ones_hbm, out_hbm, idx_d, ones_v, acc):
    c = lax.axis_index("c")
    s = lax.axis_index("s")
    wid = s * NC + c
    pltpu.sync_copy(zeros_hbm, acc.at[pl.ds(s * RT, RT)])
    pltpu.sync_copy(ones_hbm, ones_v)
    pltpu.sync_copy(dst_hbm.at[wid], idx_d)
    plsc.subcore_barrier()

    def body(j, carry):
        pltpu.sync_copy(ones_v, acc.at[idx_d.at[j]], add=True)
        return carry

    lax.fori_loop(0, CC, body, 0)
    plsc.subcore_barrier()
    pltpu.sync_copy(acc.at[pl.ds(s * RT, RT)], out_hbm.at[c, pl.ds(s * RT, RT)])


BR = 1024
GB = NP // BR


def _dense_body(aggp_ref, cntp_ref, h_ref, wl_ref, wr_ref, b_ref, o_ref, *, act):
    cnt = cntp_ref[0, :, 0:1] + cntp_ref[1, :, 0:1]
    inv = 1.0 / jnp.maximum(cnt, 1.0)
    out = (jnp.dot(aggp_ref[0] * inv, wl_ref[0], preferred_element_type=jnp.float32)
           + jnp.dot(aggp_ref[1] * inv, wl_ref[1], preferred_element_type=jnp.float32)
           + jnp.dot(h_ref[0], wr_ref[0], preferred_element_type=jnp.float32)
           + jnp.dot(h_ref[1], wr_ref[1], preferred_element_type=jnp.float32)
           + b_ref[...])
    if act == "relu":
        out = jnp.maximum(out, 0.0)
    else:
        z = out - jnp.max(out, axis=-1, keepdims=True)
        out = z - jnp.log(jnp.sum(jnp.exp(z), axis=-1, keepdims=True))
    o_ref[0] = out[:, :DH]
    o_ref[1] = out[:, DH:]


def _dense(aggp, cntp, h2, Wl2h, Wr2h, b2d, act):
    return pl.pallas_call(
        functools.partial(_dense_body, act=act),
        grid=(GB,),
        in_specs=[
            pl.BlockSpec((2, BR, DH), lambda i: (0, i, 0)),
            pl.BlockSpec((2, BR, CW), lambda i: (0, i, 0)),
            pl.BlockSpec((2, BR, DH), lambda i: (0, i, 0)),
            pl.BlockSpec((2, DH, D), lambda i: (0, 0, 0)),
            pl.BlockSpec((2, DH, D), lambda i: (0, 0, 0)),
            pl.BlockSpec((1, D), lambda i: (0, 0)),
        ],
        out_specs=pl.BlockSpec((2, BR, DH), lambda i: (0, i, 0)),
        out_shape=jax.ShapeDtypeStruct((2, NP, DH), jnp.float32),
    )(aggp, cntp, h2, Wl2h, Wr2h, b2d)


def kernel(x, edge_index, Wl1, Wr1, b1, Wl2, Wr2, b2, Wl3, Wr3, b3):
    src = edge_index[0]
    dst = edge_index[1]
    srcT = src.reshape(NS, CA, KA)
    src3 = jnp.stack([srcT, srcT + NP])
    dst3 = dst.reshape(NS, CA, KA)
    xp = jnp.zeros((NP, D), jnp.float32).at[:N].set(x)
    x2 = jnp.stack([xp[:, :DH], xp[:, DH:]])
    dstC = dst.reshape(NW, CC, KC)
    zh = jnp.zeros((RT, DH), jnp.float32)
    zc = jnp.zeros((RT, CW), jnp.float32)
    ones = jnp.ones((KC, CW), jnp.float32)

    cntp = _sc_count(dstC, zc, ones)
    h2 = x2
    for Wl, Wr, b, act in ((Wl1, Wr1, b1, "relu"),
                           (Wl2, Wr2, b2, "relu"),
                           (Wl3, Wr3, b3, "logsoftmax")):
        aggp = _sc_aggregate(h2.reshape(NC * NP, DH), src3, dst3, zh)
        h2 = _dense(aggp, cntp, h2, Wl.reshape(2, DH, D), Wr.reshape(2, DH, D),
                    b.reshape(1, D), act)
    return jnp.concatenate([h2[0], h2[1]], axis=-1)[:N]


# speedup vs baseline: 2.5292x; 1.0116x over previous
import functools

import jax
import jax.numpy as jnp
from jax import lax
from jax.experimental import pallas as pl
from jax.experimental.pallas import tpu as pltpu
from jax.experimental.pallas import tpu_sc as plsc

N = 10000
NP = 10240
E = 320000
D = 128
DH = D // 2
NC = 2
NS = 16
NW = NC * NS
RT = NP // NS

KA = 125
CA = E // NS // KA

KC = 50
CC = E // NW // KC

_MESH = plsc.VectorSubcoreMesh(core_axis_name="c", subcore_axis_name="s")


CW = 16


def _make_aggregate(with_count):
    out_type = [jax.ShapeDtypeStruct((NC, NP, DH), jnp.float32)]
    scratch = [
        pltpu.VMEM((CA, KA), jnp.int32),
        pltpu.VMEM((CA, KA), jnp.int32),
        pltpu.VMEM((KA, DH), jnp.float32),
        pltpu.VMEM((KA, DH), jnp.float32),
        pltpu.VMEM_SHARED((NP, DH), jnp.float32),
        pltpu.SemaphoreType.DMA,
        pltpu.SemaphoreType.DMA,
    ]
    if with_count:
        out_type.append(jax.ShapeDtypeStruct((NC, NP, CW), jnp.float32))
        scratch += [
            pltpu.VMEM((KA, CW), jnp.float32),
            pltpu.VMEM_SHARED((NP, CW), jnp.float32),
        ]

    def agg(h_hbm, src_hbm, dst_hbm, zeros_hbm, *rest):
        if with_count:
            (zc_hbm, ones_hbm, out_hbm, cnt_hbm,
             idx_s, idx_d, msg0, msg1, acc, sem0, sem1, ones_v, cacc) = rest
        else:
            idx_s, idx_d, msg0, msg1, acc, sem0, sem1 = rest[1:]
            out_hbm = rest[0]
        c = lax.axis_index("c")
        s = lax.axis_index("s")
        pltpu.sync_copy(zeros_hbm, acc.at[pl.ds(s * RT, RT)])
        pltpu.sync_copy(src_hbm.at[c, s], idx_s)
        pltpu.sync_copy(dst_hbm.at[s], idx_d)
        if with_count:
            pltpu.sync_copy(zc_hbm, cacc.at[pl.ds(s * RT, RT)])
            pltpu.sync_copy(ones_hbm, ones_v)
        plsc.subcore_barrier()

        pltpu.async_copy(h_hbm.at[idx_s.at[0]], msg0, sem0)
        pltpu.async_copy(h_hbm.at[idx_s.at[1]], msg1, sem1)

        def body(i, carry):
            j0 = 2 * i
            pltpu.make_async_copy(h_hbm.at[idx_s.at[j0]], msg0, sem0).wait()
            pltpu.sync_copy(msg0, acc.at[idx_d.at[j0]], add=True)
            if with_count:
                @pl.when(c == 0)
                def _():
                    pltpu.sync_copy(ones_v, cacc.at[idx_d.at[j0]], add=True)

            @pl.when(j0 + 2 < CA)
            def _():
                pltpu.async_copy(h_hbm.at[idx_s.at[j0 + 2]], msg0, sem0)

            pltpu.make_async_copy(h_hbm.at[idx_s.at[j0 + 1]], msg1, sem1).wait()
            pltpu.sync_copy(msg1, acc.at[idx_d.at[j0 + 1]], add=True)
            if with_count:
                @pl.when(c == 1)
                def _():
                    pltpu.sync_copy(ones_v, cacc.at[idx_d.at[j0 + 1]], add=True)

            @pl.when(j0 + 3 < CA)
            def _():
                pltpu.async_copy(h_hbm.at[idx_s.at[j0 + 3]], msg1, sem1)

            return carry

        lax.fori_loop(0, CA // 2, body, 0)
        plsc.subcore_barrier()
        pltpu.sync_copy(acc.at[pl.ds(s * RT, RT)], out_hbm.at[c, pl.ds(s * RT, RT)])
        if with_count:
            pltpu.sync_copy(cacc.at[pl.ds(s * RT, RT)],
                            cnt_hbm.at[c, pl.ds(s * RT, RT)])

    return functools.partial(
        pl.kernel,
        out_type=tuple(out_type) if with_count else out_type[0],
        mesh=_MESH,
        scratch_types=scratch,
        compiler_params=pltpu.CompilerParams(use_tc_tiling_on_sc=False),
    )(agg)


_sc_aggregate = _make_aggregate(False)


@functools.partial(
    pl.kernel,
    out_type=jax.ShapeDtypeStruct((NC, NP, CW), jnp.float32),
    mesh=_MESH,
    scratch_types=[
        pltpu.VMEM((CC, KC), jnp.int32),
        pltpu.VMEM((KC, CW), jnp.float32),
        pltpu.VMEM_SHARED((NP, CW), jnp.float32),
    ],
    compiler_params=pltpu.CompilerParams(use_tc_tiling_on_sc=False),
)
def _sc_count(dst_hbm, zeros_hbm, ones_hbm, out_hbm, idx_d, ones_v, acc):
    c = lax.axis_index("c")
    s = lax.axis_index("s")
    wid = s * NC + c
    pltpu.sync_copy(zeros_hbm, acc.at[pl.ds(s * RT, RT)])
    pltpu.sync_copy(ones_hbm, ones_v)
    pltpu.sync_copy(dst_hbm.at[wid], idx_d)
    plsc.subcore_barrier()

    def body(j, carry):
        pltpu.sync_copy(ones_v, acc.at[idx_d.at[j]], add=True)
        return carry

    lax.fori_loop(0, CC, body, 0)
    plsc.subcore_barrier()
    pltpu.sync_copy(acc.at[pl.ds(s * RT, RT)], out_hbm.at[c, pl.ds(s * RT, RT)])


BR = 1024
GB = NP // BR


def _dense_body(aggp_ref, cntp_ref, h_ref, wl_ref, wr_ref, b_ref, o_ref, *, act):
    cnt = cntp_ref[0, :, 0:1] + cntp_ref[1, :, 0:1]
    inv = 1.0 / jnp.maximum(cnt, 1.0)
    out = (jnp.dot(aggp_ref[0] * inv, wl_ref[0], preferred_element_type=jnp.float32)
           + jnp.dot(aggp_ref[1] * inv, wl_ref[1], preferred_element_type=jnp.float32)
           + jnp.dot(h_ref[0], wr_ref[0], preferred_element_type=jnp.float32)
           + jnp.dot(h_ref[1], wr_ref[1], preferred_element_type=jnp.float32)
           + b_ref[...])
    if act == "relu":
        out = jnp.maximum(out, 0.0)
    else:
        z = out - jnp.max(out, axis=-1, keepdims=True)
        out = z - jnp.log(jnp.sum(jnp.exp(z), axis=-1, keepdims=True))
    if o_ref.shape[0] == 2:
        o_ref[0] = out[:, :DH]
        o_ref[1] = out[:, DH:]
    else:
        o_ref[...] = out


def _dense(aggp, cntp, h2, Wl2h, Wr2h, b2d, act, split_out=True):
    if split_out:
        out_spec = pl.BlockSpec((2, BR, DH), lambda i: (0, i, 0))
        out_shape = jax.ShapeDtypeStruct((2, NP, DH), jnp.float32)
    else:
        out_spec = pl.BlockSpec((BR, D), lambda i: (i, 0))
        out_shape = jax.ShapeDtypeStruct((NP, D), jnp.float32)
    return pl.pallas_call(
        functools.partial(_dense_body, act=act),
        grid=(GB,),
        in_specs=[
            pl.BlockSpec((2, BR, DH), lambda i: (0, i, 0)),
            pl.BlockSpec((2, BR, CW), lambda i: (0, i, 0)),
            pl.BlockSpec((2, BR, DH), lambda i: (0, i, 0)),
            pl.BlockSpec((2, DH, D), lambda i: (0, 0, 0)),
            pl.BlockSpec((2, DH, D), lambda i: (0, 0, 0)),
            pl.BlockSpec((1, D), lambda i: (0, 0)),
        ],
        out_specs=out_spec,
        out_shape=out_shape,
    )(aggp, cntp, h2, Wl2h, Wr2h, b2d)


def kernel(x, edge_index, Wl1, Wr1, b1, Wl2, Wr2, b2, Wl3, Wr3, b3):
    src = edge_index[0]
    dst = edge_index[1]
    srcT = src.reshape(NS, CA, KA)
    src3 = jnp.stack([srcT, srcT + NP])
    dst3 = dst.reshape(NS, CA, KA)
    xp = jnp.zeros((NP, D), jnp.float32).at[:N].set(x)
    x2 = jnp.stack([xp[:, :DH], xp[:, DH:]])
    dstC = dst.reshape(NW, CC, KC)
    zh = jnp.zeros((RT, DH), jnp.float32)
    zc = jnp.zeros((RT, CW), jnp.float32)
    ones = jnp.ones((KC, CW), jnp.float32)

    cntp = _sc_count(dstC, zc, ones)
    h2 = x2
    for Wl, Wr, b in ((Wl1, Wr1, b1), (Wl2, Wr2, b2)):
        aggp = _sc_aggregate(h2.reshape(NC * NP, DH), src3, dst3, zh)
        h2 = _dense(aggp, cntp, h2, Wl.reshape(2, DH, D), Wr.reshape(2, DH, D),
                    b.reshape(1, D), "relu")
    aggp = _sc_aggregate(h2.reshape(NC * NP, DH), src3, dst3, zh)
    out = _dense(aggp, cntp, h2, Wl3.reshape(2, DH, D), Wr3.reshape(2, DH, D),
                 b3.reshape(1, D), "logsoftmax", split_out=False)
    return out[:N]
